# Initial kernel scaffold; baseline (speedup 1.0000x reference)
import functools

import jax
import jax.numpy as jnp
import numpy as np
from jax import lax
from jax.experimental import pallas as pl
from jax.experimental.pallas import tpu as pltpu
from jax.experimental.pallas import tpu_sc as plsc

_NUM_STATES = 32
_HIDDEN = 64
_B = 4096

_SC_TILES = 32
_SC_LANES = 16
_SC_CHUNK = _B // _SC_TILES

_ROW_BLK = 256


def _i0(x):
    t2 = (x / 3.75) * (x / 3.75)
    small = 1.0 + t2 * (3.5156229 + t2 * (3.0899424 + t2 * (1.2067492
            + t2 * (0.2659732 + t2 * (0.0360768 + t2 * 0.0045813)))))
    xl = jnp.maximum(x, 3.75)
    u = 3.75 / xl
    large = (jnp.exp(xl) / jnp.sqrt(xl)) * (0.39894228 + u * (0.01328592
            + u * (0.00225319 + u * (-0.00157565 + u * (0.00916281
            + u * (-0.02057706 + u * (0.02635537 + u * (-0.01647633
            + u * 0.00392377))))))))
    return jnp.where(x < 3.75, small, large)


def _sc_histogram(states):
    mesh = plsc.VectorSubcoreMesh(core_axis_name="c", subcore_axis_name="s")

    @functools.partial(
        pl.kernel,
        out_type=jax.ShapeDtypeStruct((_NUM_STATES, _SC_TILES * _SC_LANES),
                                      jnp.int32),
        mesh=mesh,
        scratch_types=[
            pltpu.VMEM((_SC_CHUNK,), jnp.int32),
            pltpu.VMEM((_NUM_STATES, _SC_LANES), jnp.int32),
            pltpu.SemaphoreType.DMA,
        ],
    )
    def hist_kernel(states_hbm, out_hbm, sbuf, hist, sem):
        wid = lax.axis_index("s") * 2 + lax.axis_index("c")
        pltpu.async_copy(states_hbm.at[pl.ds(wid * _SC_CHUNK, _SC_CHUNK)],
                         sbuf, sem).wait()
        vecs = [sbuf[pl.ds(r * _SC_LANES, _SC_LANES)]
                for r in range(_SC_CHUNK // _SC_LANES)]
        for b in range(_NUM_STATES):
            acc = jnp.zeros((_SC_LANES,), jnp.int32)
            for v in vecs:
                acc = acc + jnp.where(v == b, 1, 0)
            hist[b, :] = acc
        pltpu.async_copy(hist,
                         out_hbm.at[:, pl.ds(wid * _SC_LANES, _SC_LANES)],
                         sem).wait()

    return hist_kernel(states)


def _kde_body(pr_ref, pc_ref, out_ref):
    phi = pr_ref[...]
    m = jnp.sum(phi, keepdims=True) / _B
    var = jnp.sum((phi - m) ** 2, keepdims=True) / (_B - 1)
    bw = 1.06 * jnp.sqrt(var) * (_B ** -0.2)
    kap = 1.0 / (bw * bw)
    inv_den = 1.0 / (_B * 2.0 * np.pi * _i0(kap))
    cr = jnp.cos(phi)
    sr = jnp.sin(phi)

    def blk(i, acc):
        pc = pc_ref[pl.ds(i * _ROW_BLK, _ROW_BLK), :]
        ci = kap * jnp.cos(pc)
        si = kap * jnp.sin(pc)
        mm = ci * cr + si * sr
        srow = jnp.sum(jnp.exp(mm), axis=1, keepdims=True)
        p = srow * inv_den + 1e-10
        return acc + jnp.sum(jnp.log(p))

    tot = lax.fori_loop(0, _B // _ROW_BLK, blk, jnp.float32(0.0))
    out_ref[0] = -tot / _B


def _tc_kde(phi_row, phi_col):
    return pl.pallas_call(
        _kde_body,
        out_shape=jax.ShapeDtypeStruct((1,), jnp.float32),
        out_specs=pl.BlockSpec(memory_space=pltpu.SMEM),
    )(phi_row, phi_col)


def _combine_body(hist_ref, emb_ref, g_ref, be_ref, w1_ref, b1_ref,
                  wk_ref, bk_ref, hphi_ref, out_ref):
    counts = jnp.sum(hist_ref[...], axis=1, keepdims=True).astype(jnp.float32)
    probs = counts / _B + 1e-10
    h_z = -jnp.sum(probs * jnp.log(probs))

    h = emb_ref[...]
    mh = jnp.mean(h, axis=1, keepdims=True)
    vh = jnp.mean((h - mh) ** 2, axis=1, keepdims=True)
    hn = (h - mh) / jnp.sqrt(vh + 1e-5) * g_ref[...] + be_ref[...]
    h1 = jnp.maximum(hn, 0.0)
    h2 = jnp.maximum(
        jnp.dot(h1, w1_ref[...], preferred_element_type=jnp.float32)
        + b1_ref[...], 0.0)
    kp = jnp.sum(h2 * wk_ref[...], axis=1, keepdims=True) + bk_ref[...]
    kap = jnp.maximum(kp, 0.0) + jnp.log1p(jnp.exp(-jnp.abs(kp))) + 0.1
    h_vm = jnp.log(2.0 * np.pi * _i0(kap)) - kap

    h_pgz = jnp.sum((counts / _B) * h_vm)
    h_phi = hphi_ref[0]
    mi = h_phi - h_pgz
    bdc_c = jnp.clip(2.0 * mi / (h_z + h_phi + 1e-12), 0.0, 1.0)
    bdc_o = mi / jnp.minimum(h_z, h_phi)
    out_ref[0] = mi
    out_ref[1] = h_z
    out_ref[2] = h_phi
    out_ref[3] = h_pgz
    out_ref[4] = bdc_c
    out_ref[5] = bdc_o


def _tc_combine(hist, emb, ln_gamma, ln_beta, w1, b1, wk_t, bk, h_phi):
    vspec = pl.BlockSpec(memory_space=pltpu.VMEM)
    return pl.pallas_call(
        _combine_body,
        out_shape=jax.ShapeDtypeStruct((6,), jnp.float32),
        in_specs=[vspec] * 8 + [pl.BlockSpec(memory_space=pltpu.SMEM)],
        out_specs=pl.BlockSpec(memory_space=pltpu.SMEM),
    )(hist, emb, ln_gamma, ln_beta, w1, b1, wk_t, bk, h_phi)


def kernel(states, phases, emb, ln_gamma, ln_beta, W1, b1, W_mu, b_mu,
           W_k, b_k):
    del W_mu, b_mu
    hist = _sc_histogram(states)
    h_phi = _tc_kde(phases.reshape(1, _B), phases.reshape(_B, 1))
    out = _tc_combine(hist, emb, ln_gamma.reshape(1, _HIDDEN),
                      ln_beta.reshape(1, _HIDDEN), W1, b1.reshape(1, _HIDDEN),
                      W_k.reshape(1, _HIDDEN), b_k.reshape(1, 1), h_phi)
    return (out[0], out[1], out[2], out[3], out[4], out[5])


# speedup vs baseline: 5.2584x; 5.2584x over previous
import functools

import jax
import jax.numpy as jnp
import numpy as np
from jax import lax
from jax.experimental import pallas as pl
from jax.experimental.pallas import tpu as pltpu
from jax.experimental.pallas import tpu_sc as plsc

_NUM_STATES = 32
_HIDDEN = 64
_B = 4096

_SC_TILES = 32
_SC_LANES = 16
_SC_CHUNK = _B // _SC_TILES

_ROW_BLK = 256


def _i0(x):
    t2 = (x / 3.75) * (x / 3.75)
    small = 1.0 + t2 * (3.5156229 + t2 * (3.0899424 + t2 * (1.2067492
            + t2 * (0.2659732 + t2 * (0.0360768 + t2 * 0.0045813)))))
    xl = jnp.maximum(x, 3.75)
    u = 3.75 / xl
    large = (jnp.exp(xl) / jnp.sqrt(xl)) * (0.39894228 + u * (0.01328592
            + u * (0.00225319 + u * (-0.00157565 + u * (0.00916281
            + u * (-0.02057706 + u * (0.02635537 + u * (-0.01647633
            + u * 0.00392377))))))))
    return jnp.where(x < 3.75, small, large)


def _sc_histogram(states):
    mesh = plsc.VectorSubcoreMesh(core_axis_name="c", subcore_axis_name="s")

    @functools.partial(
        pl.kernel,
        out_type=jax.ShapeDtypeStruct((_SC_TILES, _NUM_STATES, _SC_LANES),
                                      jnp.int32),
        mesh=mesh,
        scratch_types=[
            pltpu.VMEM((_SC_CHUNK,), jnp.int32),
            pltpu.VMEM((_NUM_STATES, _SC_LANES), jnp.int32),
            pltpu.SemaphoreType.DMA,
        ],
    )
    def hist_kernel(states_hbm, out_hbm, sbuf, hist, sem):
        wid = lax.axis_index("s") * 2 + lax.axis_index("c")
        pltpu.async_copy(states_hbm.at[pl.ds(wid * _SC_CHUNK, _SC_CHUNK)],
                         sbuf, sem).wait()
        vecs = [sbuf[pl.ds(r * _SC_LANES, _SC_LANES)]
                for r in range(_SC_CHUNK // _SC_LANES)]
        for b in range(_NUM_STATES):
            acc = jnp.zeros((_SC_LANES,), jnp.int32)
            for v in vecs:
                acc = acc + jnp.where(v == b, 1, 0)
            hist[b, :] = acc
        pltpu.async_copy(hist, out_hbm.at[wid], sem).wait()

    return hist_kernel(states)


def _kde_body(pr_ref, pc_ref, out_ref):
    phi = pr_ref[...]
    m = jnp.sum(phi, keepdims=True) / _B
    var = jnp.sum((phi - m) ** 2, keepdims=True) / (_B - 1)
    bw = 1.06 * jnp.sqrt(var) * (_B ** -0.2)
    kap = 1.0 / (bw * bw)
    inv_den = 1.0 / (_B * 2.0 * np.pi * _i0(kap))
    cr = jnp.cos(phi)
    sr = jnp.sin(phi)

    def blk(i, acc):
        pc = pc_ref[pl.ds(i * _ROW_BLK, _ROW_BLK), :]
        ci = kap * jnp.cos(pc)
        si = kap * jnp.sin(pc)
        mm = ci * cr + si * sr
        srow = jnp.sum(jnp.exp(mm), axis=1, keepdims=True)
        p = srow * inv_den + 1e-10
        return acc + jnp.sum(jnp.log(p))

    tot = lax.fori_loop(0, _B // _ROW_BLK, blk, jnp.float32(0.0))
    out_ref[0] = -tot / _B


def _tc_kde(phi_row, phi_col):
    return pl.pallas_call(
        _kde_body,
        out_shape=jax.ShapeDtypeStruct((1,), jnp.float32),
        out_specs=pl.BlockSpec(memory_space=pltpu.SMEM),
    )(phi_row, phi_col)


def _combine_body(hist_ref, emb_ref, g_ref, be_ref, w1_ref, b1_ref,
                  wk_ref, bk_ref, hphi_ref, out_ref):
    part = jnp.sum(hist_ref[...], axis=0)
    counts = jnp.sum(part, axis=1, keepdims=True).astype(jnp.float32)
    probs = counts / _B + 1e-10
    h_z = -jnp.sum(probs * jnp.log(probs))

    h = emb_ref[...]
    mh = jnp.mean(h, axis=1, keepdims=True)
    vh = jnp.mean((h - mh) ** 2, axis=1, keepdims=True)
    hn = (h - mh) / jnp.sqrt(vh + 1e-5) * g_ref[...] + be_ref[...]
    h1 = jnp.maximum(hn, 0.0)
    h2 = jnp.maximum(
        jnp.dot(h1, w1_ref[...], preferred_element_type=jnp.float32)
        + b1_ref[...], 0.0)
    kp = jnp.sum(h2 * wk_ref[...], axis=1, keepdims=True) + bk_ref[...]
    kap = jnp.maximum(kp, 0.0) + jnp.log1p(jnp.exp(-jnp.abs(kp))) + 0.1
    h_vm = jnp.log(2.0 * np.pi * _i0(kap)) - kap

    h_pgz = jnp.sum((counts / _B) * h_vm)
    h_phi = hphi_ref[0]
    mi = h_phi - h_pgz
    bdc_c = jnp.clip(2.0 * mi / (h_z + h_phi + 1e-12), 0.0, 1.0)
    bdc_o = mi / jnp.minimum(h_z, h_phi)
    out_ref[0] = mi
    out_ref[1] = h_z
    out_ref[2] = h_phi
    out_ref[3] = h_pgz
    out_ref[4] = bdc_c
    out_ref[5] = bdc_o


def _tc_combine(hist, emb, ln_gamma, ln_beta, w1, b1, wk_t, bk, h_phi):
    vspec = pl.BlockSpec(memory_space=pltpu.VMEM)
    return pl.pallas_call(
        _combine_body,
        out_shape=jax.ShapeDtypeStruct((6,), jnp.float32),
        in_specs=[vspec] * 8 + [pl.BlockSpec(memory_space=pltpu.SMEM)],
        out_specs=pl.BlockSpec(memory_space=pltpu.SMEM),
    )(hist, emb, ln_gamma, ln_beta, w1, b1, wk_t, bk, h_phi)


def kernel(states, phases, emb, ln_gamma, ln_beta, W1, b1, W_mu, b_mu,
           W_k, b_k):
    del W_mu, b_mu
    hist = _sc_histogram(states)
    h_phi = _tc_kde(phases.reshape(1, _B), phases.reshape(_B, 1))
    out = _tc_combine(hist, emb, ln_gamma.reshape(1, _HIDDEN),
                      ln_beta.reshape(1, _HIDDEN), W1, b1.reshape(1, _HIDDEN),
                      W_k.reshape(1, _HIDDEN), b_k.reshape(1, 1), h_phi)
    return (out[0], out[1], out[2], out[3], out[4], out[5])
